# baseline (speedup 1.0000x reference)
import jax
import jax.numpy as jnp
from jax import lax
from jax.experimental import pallas as pl
from jax.experimental.pallas import tpu as pltpu
from jax.experimental.pallas import tpu_sc as plsc

N = 10000
E = 320000
D_IN = 128
D_HID = 128
D_OUT = 40
D_OUT_PAD = 48

NC = 2
NS = 16
NW = NC * NS
C = 128
N_PAD = 10240
RPT = N_PAD // NS

EPW1 = 20224
NCH1 = EPW1 // C
E_PAD1 = NS * EPW1
DH = D_HID // NC

EPW2 = 10240
NCH2 = EPW2 // C
E_PAD2 = NW * EPW2


def _zero_spmem(rows, w_l, acc_sh, den_sh, sid, dv):
    zv = jnp.zeros((16,), jnp.float32)

    @plsc.parallel_loop(0, C, step=1, unroll=8)
    def _(r):
        for j in range(dv):
            rows[r, pl.ds(j * 16, 16)] = zv
    for j in range(C // 16):
        w_l[0, pl.ds(j * 16, 16)] = zv
    for k in range(RPT // C):
        pltpu.sync_copy(rows, acc_sh.at[pl.ds(sid * RPT + k * C, C)])
        pltpu.sync_copy(w_l.at[0], den_sh.at[pl.ds(sid * RPT + k * C, C)])


def _fused_pipeline(h_hbm, src_l, dst_l, w_c, bufs, sems, acc_sh, den_sh,
                    asrc_l, adst_l, nch, dv, den_pred, base, off):
    zi = jnp.zeros((16,), jnp.int32)
    offv = zi + off

    def g_src(ch):
        return h_hbm.at[src_l.at[pl.ds(ch * C, C)]]

    pltpu.async_copy(g_src(0), bufs[0], sems[0])

    def outer(g2, _):
        g = g2 * 2
        for b in range(2):
            ch = g + b
            buf = bufs[b]

            @pl.when(ch + 1 < nch)
            def _():
                pltpu.async_copy(g_src(ch + 1), bufs[1 - b], sems[1 - b])

            @plsc.parallel_loop(0, C // 16, step=1, unroll=4)
            def _(k):
                s_idx = src_l[pl.ds(ch * C + k * 16, 16)] - offv
                d_idx = dst_l[ch, pl.ds(k * 16, 16)]
                sv = plsc.load_gather(asrc_l, [s_idx])
                dvv = plsc.load_gather(adst_l, [d_idx])
                e = sv + dvv
                e = jnp.where(e > 0.0, e, 0.2 * e)
                w = jnp.exp(e)
                gid = base + ch * C + k * 16 + lax.iota(jnp.int32, 16)
                w = jnp.where(gid < E, w, 0.0)
                w_c[b, pl.ds(k * 16, 16)] = w

            pltpu.make_async_copy(g_src(ch), buf, sems[b]).wait()

            @plsc.parallel_loop(0, C, step=1, unroll=4)
            def _(i):
                wv = plsc.load_gather(w_c, [zi + b, zi + i])
                for j in range(dv):
                    buf[i, pl.ds(j * 16, 16)] = buf[i, pl.ds(j * 16, 16)] * wv

            pltpu.sync_copy(buf, acc_sh.at[dst_l.at[ch]], add=True)

            @pl.when(den_pred)
            def _():
                pltpu.sync_copy(w_c.at[b], den_sh.at[dst_l.at[ch]], add=True)
        return 0

    lax.fori_loop(0, nch // 2, outer, 0)


def _sc_layer1():
    def body(h_hbm, asrc_hbm, adst_hbm, src_hbm, dst_hbm,
             acc_out, den_out,
             asrc_l, adst_l, src_l, dst_l, w_c, rows, rows2,
             acc_sh, den_sh, sem, sem2):
        cid = lax.axis_index("c")
        sid = lax.axis_index("s")

        _zero_spmem(rows, w_c, acc_sh, den_sh, sid, DH // 16)
        plsc.subcore_barrier()

        pltpu.sync_copy(asrc_hbm, asrc_l)
        pltpu.sync_copy(adst_hbm, adst_l)
        pltpu.sync_copy(src_hbm.at[pl.ds(sid * EPW1, EPW1)], src_l)
        pltpu.sync_copy(dst_hbm.at[sid], dst_l)

        off = jnp.zeros((16,), jnp.int32) + cid * N

        @plsc.parallel_loop(0, EPW1 // 16, step=1, unroll=8)
        def _(i):
            src_l[pl.ds(i * 16, 16)] = src_l[pl.ds(i * 16, 16)] + off

        _fused_pipeline(h_hbm, src_l, dst_l, w_c, (rows, rows2), (sem, sem2),
                        acc_sh, den_sh, asrc_l, adst_l, NCH1, DH // 16,
                        cid == 0, sid * EPW1, cid * N)

        plsc.subcore_barrier()
        pltpu.sync_copy(acc_sh.at[pl.ds(sid * RPT, RPT)],
                        acc_out.at[cid, pl.ds(sid * RPT, RPT)])
        @pl.when(cid == 0)
        def _():
            pltpu.sync_copy(den_sh.at[pl.ds(sid * RPT, RPT)],
                            den_out.at[pl.ds(sid * RPT, RPT)])

    mesh = plsc.VectorSubcoreMesh(core_axis_name="c", subcore_axis_name="s")
    return pl.kernel(
        body,
        out_type=[
            jax.ShapeDtypeStruct((NC, N_PAD, DH), jnp.float32),
            jax.ShapeDtypeStruct((N_PAD,), jnp.float32),
        ],
        mesh=mesh,
        compiler_params=pltpu.CompilerParams(needs_layout_passes=False, use_tc_tiling_on_sc=False),
        scratch_types=[
            pltpu.VMEM((N,), jnp.float32),
            pltpu.VMEM((N,), jnp.float32),
            pltpu.VMEM((EPW1,), jnp.int32),
            pltpu.VMEM((NCH1, C), jnp.int32),
            pltpu.VMEM((2, C), jnp.float32),
            pltpu.VMEM((C, DH), jnp.float32),
            pltpu.VMEM((C, DH), jnp.float32),
            pltpu.VMEM_SHARED((N_PAD, DH), jnp.float32),
            pltpu.VMEM_SHARED((N_PAD,), jnp.float32),
            pltpu.SemaphoreType.DMA,
            pltpu.SemaphoreType.DMA,
        ],
    )


def _sc_layer2():
    d = D_OUT_PAD

    def body(h_hbm, asrc_hbm, adst_hbm, src_hbm, dst_hbm,
             acc_out, den_out,
             asrc_l, adst_l, src_l, dst_l, w_c, rows, rows2, acc_sh, den_sh,
             sem, sem2):
        cid = lax.axis_index("c")
        sid = lax.axis_index("s")
        wid = sid * NC + cid
        dv = d // 16

        _zero_spmem(rows, w_c, acc_sh, den_sh, sid, dv)
        plsc.subcore_barrier()

        pltpu.sync_copy(asrc_hbm, asrc_l)
        pltpu.sync_copy(adst_hbm, adst_l)
        pltpu.sync_copy(src_hbm.at[pl.ds(wid * EPW2, EPW2)], src_l)
        pltpu.sync_copy(dst_hbm.at[wid], dst_l)

        off = jnp.zeros((16,), jnp.int32) + cid * N

        @plsc.parallel_loop(0, EPW2 // 16, step=1, unroll=8)
        def _(i):
            src_l[pl.ds(i * 16, 16)] = src_l[pl.ds(i * 16, 16)] + off

        _fused_pipeline(h_hbm, src_l, dst_l, w_c, (rows, rows2), (sem, sem2),
                        acc_sh, den_sh, asrc_l, adst_l, NCH2, dv,
                        cid >= 0, wid * EPW2, cid * N)

        plsc.subcore_barrier()
        pltpu.sync_copy(acc_sh.at[pl.ds(sid * RPT, RPT)],
                        acc_out.at[cid, pl.ds(sid * RPT, RPT)])
        pltpu.sync_copy(den_sh.at[pl.ds(sid * RPT, RPT)],
                        den_out.at[cid, pl.ds(sid * RPT, RPT)])

    mesh = plsc.VectorSubcoreMesh(core_axis_name="c", subcore_axis_name="s")
    return pl.kernel(
        body,
        out_type=[
            jax.ShapeDtypeStruct((NC, N_PAD, d), jnp.float32),
            jax.ShapeDtypeStruct((NC, N_PAD), jnp.float32),
        ],
        mesh=mesh,
        compiler_params=pltpu.CompilerParams(needs_layout_passes=False, use_tc_tiling_on_sc=False),
        scratch_types=[
            pltpu.VMEM((N,), jnp.float32),
            pltpu.VMEM((N,), jnp.float32),
            pltpu.VMEM((EPW2,), jnp.int32),
            pltpu.VMEM((NCH2, C), jnp.int32),
            pltpu.VMEM((2, C), jnp.float32),
            pltpu.VMEM((C, d), jnp.float32),
            pltpu.VMEM((C, d), jnp.float32),
            pltpu.VMEM_SHARED((N_PAD, d), jnp.float32),
            pltpu.VMEM_SHARED((N_PAD,), jnp.float32),
            pltpu.SemaphoreType.DMA,
            pltpu.SemaphoreType.DMA,
        ],
    )


def _tc_layer0(x, W1, A1):
    B = 2000

    def body(x_ref, w_ref, a_ref, h_ref, p_ref):
        h = jnp.dot(x_ref[...], w_ref[...], preferred_element_type=jnp.float32)
        h_ref[...] = h
        p_ref[...] = jnp.dot(h, a_ref[...], preferred_element_type=jnp.float32)

    return pl.pallas_call(
        body,
        grid=(N // B,),
        in_specs=[
            pl.BlockSpec((B, D_IN), lambda i: (i, 0)),
            pl.BlockSpec((D_IN, D_HID), lambda i: (0, 0)),
            pl.BlockSpec((D_HID, 2), lambda i: (0, 0)),
        ],
        out_specs=[
            pl.BlockSpec((B, D_HID), lambda i: (i, 0)),
            pl.BlockSpec((B, 2), lambda i: (i, 0)),
        ],
        out_shape=[
            jax.ShapeDtypeStruct((N, D_HID), jnp.float32),
            jax.ShapeDtypeStruct((N, 2), jnp.float32),
        ],
    )(x, W1, A1)


def _tc_mid(accL, accR, den, proj1, h1, W2p, A2p):
    B = 2000

    def body(aL, aR, d0, p1, h1r, w2, a2, h2_ref, p2_ref):
        e = p1[:, 0:1] + p1[:, 1:2]
        wself = jnp.exp(jnp.where(e > 0.0, e, 0.2 * e))
        a0 = jnp.concatenate([aL[...], aR[...]], axis=1)
        num = a0 + wself * h1r[...]
        den_ = d0[...] + wself
        out1 = num / den_
        h1a = jnp.maximum(out1, 0.0)
        h2 = jnp.dot(h1a, w2[...], preferred_element_type=jnp.float32)
        h2_ref[...] = h2
        p2_ref[...] = jnp.dot(h2, a2[...], preferred_element_type=jnp.float32)

    return pl.pallas_call(
        body,
        grid=(N // B,),
        in_specs=[
            pl.BlockSpec((B, DH), lambda i: (i, 0)),
            pl.BlockSpec((B, DH), lambda i: (i, 0)),
            pl.BlockSpec((B, 1), lambda i: (i, 0)),
            pl.BlockSpec((B, 2), lambda i: (i, 0)),
            pl.BlockSpec((B, D_HID), lambda i: (i, 0)),
            pl.BlockSpec((D_HID, D_OUT_PAD), lambda i: (0, 0)),
            pl.BlockSpec((D_OUT_PAD, 2), lambda i: (0, 0)),
        ],
        out_specs=[
            pl.BlockSpec((B, D_OUT_PAD), lambda i: (i, 0)),
            pl.BlockSpec((B, 2), lambda i: (i, 0)),
        ],
        out_shape=[
            jax.ShapeDtypeStruct((N, D_OUT_PAD), jnp.float32),
            jax.ShapeDtypeStruct((N, 2), jnp.float32),
        ],
    )(accL, accR, den, proj1, h1, W2p, A2p)


def _tc_final(acc0, acc1, den0, den1, proj2, h2p):
    B = 2000

    def body(a0, a1, d0, d1, p2, h2r, out_ref):
        e = p2[:, 0:1] + p2[:, 1:2]
        wself = jnp.exp(jnp.where(e > 0.0, e, 0.2 * e))
        num = a0[...] + a1[...] + wself * h2r[...]
        den = d0[...] + d1[...] + wself
        out_ref[...] = num / den

    return pl.pallas_call(
        body,
        grid=(N // B,),
        in_specs=[
            pl.BlockSpec((B, D_OUT_PAD), lambda i: (i, 0)),
            pl.BlockSpec((B, D_OUT_PAD), lambda i: (i, 0)),
            pl.BlockSpec((B, 1), lambda i: (i, 0)),
            pl.BlockSpec((B, 1), lambda i: (i, 0)),
            pl.BlockSpec((B, 2), lambda i: (i, 0)),
            pl.BlockSpec((B, D_OUT_PAD), lambda i: (i, 0)),
        ],
        out_specs=pl.BlockSpec((B, D_OUT_PAD), lambda i: (i, 0)),
        out_shape=jax.ShapeDtypeStruct((N, D_OUT_PAD), jnp.float32),
    )(acc0, acc1, den0, den1, proj2, h2p)


def kernel(x, edge_index, W1, a_src1, a_dst1, W2, a_src2, a_dst2):
    src = edge_index[0]
    dst = edge_index[1]
    pad1 = jnp.zeros((E_PAD1 - E,), jnp.int32)
    src1 = jnp.concatenate([src, pad1])
    dst1 = jnp.concatenate([dst, pad1]).reshape(NS, NCH1, C)
    pad2 = jnp.zeros((E_PAD2 - E,), jnp.int32)
    src2 = jnp.concatenate([src, pad2])
    dst2 = jnp.concatenate([dst, pad2]).reshape(NW, NCH2, C)

    A1 = jnp.stack([a_src1, a_dst1], axis=1)
    W2p = jnp.pad(W2, ((0, 0), (0, D_OUT_PAD - D_OUT)))
    A2 = jnp.stack([a_src2, a_dst2], axis=1)
    A2p = jnp.pad(A2, ((0, D_OUT_PAD - D_OUT), (0, 0)))

    h1, proj1 = _tc_layer0(x, W1, A1)
    h1stack = jnp.concatenate([h1[:, :DH], h1[:, DH:]], axis=0)
    a1s = proj1[:, 0]
    a1d = proj1[:, 1]
    acc1, den1 = _sc_layer1()(h1stack, a1s, a1d, src1, dst1)
    h2p, proj2 = _tc_mid(acc1[0, :N], acc1[1, :N], den1[:N, None],
                         proj1, h1, W2p, A2p)

    a2s = proj2[:, 0]
    a2d = proj2[:, 1]
    h2p2 = jnp.concatenate([h2p, h2p], axis=0)
    acc2, den2 = _sc_layer2()(h2p2, a2s, a2d, src2, dst2)
    outp = _tc_final(
        acc2[0, :N], acc2[1, :N],
        den2[0, :N, None], den2[1, :N, None],
        proj2, h2p)
    return outp[:, :D_OUT]


# speedup vs baseline: 1.0353x; 1.0353x over previous
import jax
import jax.numpy as jnp
from jax import lax
from jax.experimental import pallas as pl
from jax.experimental.pallas import tpu as pltpu
from jax.experimental.pallas import tpu_sc as plsc

N = 10000
E = 320000
D_IN = 128
D_HID = 128
D_OUT = 40
D_OUT_PAD = 48

NC = 2
NS = 16
NW = NC * NS
C = 128
N_PAD = 10240
RPT = N_PAD // NS

EPW1 = 20224
NCH1 = EPW1 // C
E_PAD1 = NS * EPW1
DH = D_HID // NC

EPW2 = 10240
NCH2 = EPW2 // C
E_PAD2 = NW * EPW2


def _zero_spmem(rows, w_l, acc_sh, den_sh, sid, dv):
    zv = jnp.zeros((16,), jnp.float32)

    @plsc.parallel_loop(0, C, step=1, unroll=8)
    def _(r):
        for j in range(dv):
            rows[r, pl.ds(j * 16, 16)] = zv
    for j in range(C // 16):
        w_l[0, pl.ds(j * 16, 16)] = zv
    for k in range(RPT // C):
        pltpu.sync_copy(rows, acc_sh.at[pl.ds(sid * RPT + k * C, C)])
        pltpu.sync_copy(w_l.at[0], den_sh.at[pl.ds(sid * RPT + k * C, C)])


def _fused_pipeline(h_hbm, src_l, dst_l, w_c, bufs, sems, acc_sh, den_sh,
                    asrc_l, adst_l, nch, dv, den_pred, base, off):
    zi = jnp.zeros((16,), jnp.int32)
    offv = zi + off

    def g_src(ch):
        return h_hbm.at[src_l.at[pl.ds(ch * C, C)]]

    pltpu.async_copy(g_src(0), bufs[0], sems[0])

    def outer(g2, _):
        g = g2 * 2
        for b in range(2):
            ch = g + b
            buf = bufs[b]

            @pl.when(ch + 1 < nch)
            def _():
                pltpu.async_copy(g_src(ch + 1), bufs[1 - b], sems[1 - b])

            @plsc.parallel_loop(0, C // 16, step=1, unroll=4)
            def _(k):
                s_idx = src_l[pl.ds(ch * C + k * 16, 16)] - offv
                d_idx = dst_l[ch, pl.ds(k * 16, 16)]
                sv = plsc.load_gather(asrc_l, [s_idx])
                dvv = plsc.load_gather(adst_l, [d_idx])
                e = sv + dvv
                e = jnp.where(e > 0.0, e, 0.2 * e)
                w = jnp.exp(e)
                gid = base + ch * C + k * 16 + lax.iota(jnp.int32, 16)
                w = jnp.where(gid < E, w, 0.0)
                w_c[b, pl.ds(k * 16, 16)] = w

            pltpu.make_async_copy(g_src(ch), buf, sems[b]).wait()

            @plsc.parallel_loop(0, C, step=1, unroll=4)
            def _(i):
                wv = plsc.load_gather(w_c, [zi + b, zi + i])
                for j in range(dv):
                    buf[i, pl.ds(j * 16, 16)] = buf[i, pl.ds(j * 16, 16)] * wv

            pltpu.sync_copy(buf, acc_sh.at[dst_l.at[ch]], add=True)

            @pl.when(den_pred)
            def _():
                pltpu.sync_copy(w_c.at[b], den_sh.at[dst_l.at[ch]], add=True)
        return 0

    lax.fori_loop(0, nch // 2, outer, 0)


def _sc_layer1():
    def body(h_hbm, asrc_hbm, adst_hbm, src_hbm, dst_hbm,
             acc_out, den_out,
             asrc_l, adst_l, src_l, dst_l, w_c, rows, rows2,
             acc_sh, den_sh, sem, sem2):
        cid = lax.axis_index("c")
        sid = lax.axis_index("s")

        _zero_spmem(rows, w_c, acc_sh, den_sh, sid, DH // 16)
        plsc.subcore_barrier()

        pltpu.sync_copy(asrc_hbm, asrc_l)
        pltpu.sync_copy(adst_hbm, adst_l)
        pltpu.sync_copy(src_hbm.at[pl.ds(sid * EPW1, EPW1)], src_l)
        pltpu.sync_copy(dst_hbm.at[sid], dst_l)

        off = jnp.zeros((16,), jnp.int32) + cid * N

        @plsc.parallel_loop(0, EPW1 // 16, step=1, unroll=8)
        def _(i):
            src_l[pl.ds(i * 16, 16)] = src_l[pl.ds(i * 16, 16)] + off

        _fused_pipeline(h_hbm, src_l, dst_l, w_c, (rows, rows2), (sem, sem2),
                        acc_sh, den_sh, asrc_l, adst_l, NCH1, DH // 16,
                        cid == 0, sid * EPW1, cid * N)

        plsc.subcore_barrier()
        pltpu.sync_copy(acc_sh.at[pl.ds(sid * RPT, RPT)],
                        acc_out.at[cid, pl.ds(sid * RPT, RPT)])
        @pl.when(cid == 0)
        def _():
            pltpu.sync_copy(den_sh.at[pl.ds(sid * RPT, RPT)],
                            den_out.at[pl.ds(sid * RPT, RPT)])

    mesh = plsc.VectorSubcoreMesh(core_axis_name="c", subcore_axis_name="s")
    return pl.kernel(
        body,
        out_type=[
            jax.ShapeDtypeStruct((NC, N_PAD, DH), jnp.float32),
            jax.ShapeDtypeStruct((N_PAD,), jnp.float32),
        ],
        mesh=mesh,
        compiler_params=pltpu.CompilerParams(needs_layout_passes=False, use_tc_tiling_on_sc=False),
        scratch_types=[
            pltpu.VMEM((N,), jnp.float32),
            pltpu.VMEM((N,), jnp.float32),
            pltpu.VMEM((EPW1,), jnp.int32),
            pltpu.VMEM((NCH1, C), jnp.int32),
            pltpu.VMEM((2, C), jnp.float32),
            pltpu.VMEM((C, DH), jnp.float32),
            pltpu.VMEM((C, DH), jnp.float32),
            pltpu.VMEM_SHARED((N_PAD, DH), jnp.float32),
            pltpu.VMEM_SHARED((N_PAD,), jnp.float32),
            pltpu.SemaphoreType.DMA,
            pltpu.SemaphoreType.DMA,
        ],
    )


def _sc_layer2():
    d = D_OUT_PAD

    def body(h_hbm, asrc_hbm, adst_hbm, src_hbm, dst_hbm,
             acc_out, den_out,
             asrc_l, adst_l, src_l, dst_l, w_c, rows, rows2, acc_sh, den_sh,
             sem, sem2):
        cid = lax.axis_index("c")
        sid = lax.axis_index("s")
        wid = sid * NC + cid
        dv = d // 16

        _zero_spmem(rows, w_c, acc_sh, den_sh, sid, dv)
        plsc.subcore_barrier()

        pltpu.sync_copy(asrc_hbm, asrc_l)
        pltpu.sync_copy(adst_hbm, adst_l)
        pltpu.sync_copy(src_hbm.at[pl.ds(wid * EPW2, EPW2)], src_l)
        pltpu.sync_copy(dst_hbm.at[wid], dst_l)

        _fused_pipeline(h_hbm, src_l, dst_l, w_c, (rows, rows2), (sem, sem2),
                        acc_sh, den_sh, asrc_l, adst_l, NCH2, dv,
                        cid >= 0, wid * EPW2, 0)

        plsc.subcore_barrier()
        pltpu.sync_copy(acc_sh.at[pl.ds(sid * RPT, RPT)],
                        acc_out.at[cid, pl.ds(sid * RPT, RPT)])
        pltpu.sync_copy(den_sh.at[pl.ds(sid * RPT, RPT)],
                        den_out.at[cid, pl.ds(sid * RPT, RPT)])

    mesh = plsc.VectorSubcoreMesh(core_axis_name="c", subcore_axis_name="s")
    return pl.kernel(
        body,
        out_type=[
            jax.ShapeDtypeStruct((NC, N_PAD, d), jnp.float32),
            jax.ShapeDtypeStruct((NC, N_PAD), jnp.float32),
        ],
        mesh=mesh,
        compiler_params=pltpu.CompilerParams(needs_layout_passes=False, use_tc_tiling_on_sc=False),
        scratch_types=[
            pltpu.VMEM((N,), jnp.float32),
            pltpu.VMEM((N,), jnp.float32),
            pltpu.VMEM((EPW2,), jnp.int32),
            pltpu.VMEM((NCH2, C), jnp.int32),
            pltpu.VMEM((2, C), jnp.float32),
            pltpu.VMEM((C, d), jnp.float32),
            pltpu.VMEM((C, d), jnp.float32),
            pltpu.VMEM_SHARED((N_PAD, d), jnp.float32),
            pltpu.VMEM_SHARED((N_PAD,), jnp.float32),
            pltpu.SemaphoreType.DMA,
            pltpu.SemaphoreType.DMA,
        ],
    )


def _tc_layer0(x, W1, A1):
    B = 2000

    def body(x_ref, w_ref, a_ref, h_ref, p_ref):
        j = pl.program_id(1)
        h = jnp.dot(x_ref[...], w_ref[0], preferred_element_type=jnp.float32)
        h_ref[...] = h
        p = jnp.dot(h, a_ref[...], preferred_element_type=jnp.float32)

        @pl.when(j == 0)
        def _():
            p_ref[...] = p

        @pl.when(j == 1)
        def _():
            p_ref[...] = p_ref[...] + p

    return pl.pallas_call(
        body,
        grid=(N // B, 2),
        in_specs=[
            pl.BlockSpec((B, D_IN), lambda i, j: (i, 0)),
            pl.BlockSpec((1, D_IN, DH), lambda i, j: (j, 0, 0)),
            pl.BlockSpec((DH, 2), lambda i, j: (j, 0)),
        ],
        out_specs=[
            pl.BlockSpec((B, DH), lambda i, j: (j * (N // B) + i, 0)),
            pl.BlockSpec((B, 2), lambda i, j: (i, 0)),
        ],
        out_shape=[
            jax.ShapeDtypeStruct((2 * N, DH), jnp.float32),
            jax.ShapeDtypeStruct((N, 2), jnp.float32),
        ],
    )(x, W1.reshape(D_IN, 2, DH).transpose(1, 0, 2), A1)


def _tc_mid(acc, den, proj1, h2t, W2p, A2p):
    B = 2000

    def body(a_ref, d0, p1, hL, hR, w2, a2, h2_ref, p2_ref):
        e = p1[:, 0:1] + p1[:, 1:2]
        wself = jnp.exp(jnp.where(e > 0.0, e, 0.2 * e))
        a0 = jnp.concatenate([a_ref[0], a_ref[1]], axis=1)
        h1r = jnp.concatenate([hL[...], hR[...]], axis=1)
        num = a0 + wself * h1r
        den_ = d0[...] + wself
        out1 = num / den_
        h1a = jnp.maximum(out1, 0.0)
        h2 = jnp.dot(h1a, w2[...], preferred_element_type=jnp.float32)
        h2_ref[...] = h2
        p2_ref[...] = jnp.dot(h2, a2[...], preferred_element_type=jnp.float32)

    return pl.pallas_call(
        body,
        grid=(N // B,),
        in_specs=[
            pl.BlockSpec((2, B, DH), lambda i: (0, i, 0)),
            pl.BlockSpec((B, 1), lambda i: (i, 0)),
            pl.BlockSpec((B, 2), lambda i: (i, 0)),
            pl.BlockSpec((B, DH), lambda i: (i, 0)),
            pl.BlockSpec((B, DH), lambda i: (N // B + i, 0)),
            pl.BlockSpec((D_HID, D_OUT_PAD), lambda i: (0, 0)),
            pl.BlockSpec((D_OUT_PAD, 2), lambda i: (0, 0)),
        ],
        out_specs=[
            pl.BlockSpec((B, D_OUT_PAD), lambda i: (i, 0)),
            pl.BlockSpec((B, 2), lambda i: (i, 0)),
        ],
        out_shape=[
            jax.ShapeDtypeStruct((N, D_OUT_PAD), jnp.float32),
            jax.ShapeDtypeStruct((N, 2), jnp.float32),
        ],
    )(acc, den, proj1, h2t, h2t, W2p, A2p)


def _tc_final(acc, den, proj2, h2p):
    B = 2000

    def body(a_ref, d_ref, p2, h2r, out_ref):
        e = p2[:, 0:1] + p2[:, 1:2]
        wself = jnp.exp(jnp.where(e > 0.0, e, 0.2 * e))
        num = a_ref[0] + a_ref[1] + wself * h2r[...]
        den_ = d_ref[0] + d_ref[1] + wself
        out_ref[...] = num / den_

    return pl.pallas_call(
        body,
        grid=(N // B,),
        in_specs=[
            pl.BlockSpec((2, B, D_OUT_PAD), lambda i: (0, i, 0)),
            pl.BlockSpec((2, B, 1), lambda i: (0, i, 0)),
            pl.BlockSpec((B, 2), lambda i: (i, 0)),
            pl.BlockSpec((B, D_OUT_PAD), lambda i: (i, 0)),
        ],
        out_specs=pl.BlockSpec((B, D_OUT_PAD), lambda i: (i, 0)),
        out_shape=jax.ShapeDtypeStruct((N, D_OUT_PAD), jnp.float32),
    )(acc, den, proj2, h2p)


def kernel(x, edge_index, W1, a_src1, a_dst1, W2, a_src2, a_dst2):
    src = edge_index[0]
    dst = edge_index[1]
    pad1 = jnp.zeros((E_PAD1 - E,), jnp.int32)
    src1 = jnp.concatenate([src, pad1])
    dst1 = jnp.concatenate([dst, pad1]).reshape(NS, NCH1, C)
    pad2 = jnp.zeros((E_PAD2 - E,), jnp.int32)
    src2 = jnp.concatenate([src, pad2])
    dst2 = jnp.concatenate([dst, pad2]).reshape(NW, NCH2, C)

    A1 = jnp.stack([a_src1, a_dst1], axis=1)
    W2p = jnp.pad(W2, ((0, 0), (0, D_OUT_PAD - D_OUT)))
    A2 = jnp.stack([a_src2, a_dst2], axis=1)
    A2p = jnp.pad(A2, ((0, D_OUT_PAD - D_OUT), (0, 0)))

    h2t, proj1 = _tc_layer0(x, W1, A1)
    a1s = proj1[:, 0]
    a1d = proj1[:, 1]
    acc1, den1 = _sc_layer1()(h2t, a1s, a1d, src1, dst1)
    h2p, proj2 = _tc_mid(acc1, den1[:, None], proj1, h2t, W2p, A2p)

    a2s = proj2[:, 0]
    a2d = proj2[:, 1]
    acc2, den2 = _sc_layer2()(h2p, a2s, a2d, src2, dst2)
    outp = _tc_final(acc2, den2[:, :, None], proj2, h2p)
    return outp[:, :D_OUT]
